# Initial kernel scaffold; baseline (speedup 1.0000x reference)
import jax
import jax.numpy as jnp
from jax.experimental import pallas as pl


def kernel(feat, loc, W1, b1, W2, b2):
    raise NotImplementedError("write your pallas kernel here")


# speedup vs baseline: 12.1329x; 12.1329x over previous
import functools

import jax
import jax.numpy as jnp
from jax import lax
from jax.experimental import pallas as pl
from jax.experimental.pallas import tpu as pltpu
from jax.experimental.pallas import tpu_sc as plsc

_K = 32


def _fps_body(loc_ref, setloc_ref, dists_ref, ax_ref, ay_ref, az_ref):
    b, _, n = loc_ref.shape
    m = setloc_ref.shape[-1]
    colid = lax.broadcasted_iota(jnp.int32, (b, n), 1)
    colm = lax.broadcasted_iota(jnp.int32, (b, m), 1)
    dists_ref[...] = jnp.full((b, n), jnp.inf, jnp.float32)
    lx0 = loc_ref[:, 0, 0:1]
    ly0 = loc_ref[:, 1, 0:1]
    lz0 = loc_ref[:, 2, 0:1]
    first = colm == 0
    ax_ref[...] = jnp.where(first, lx0, 0.0)
    ay_ref[...] = jnp.where(first, ly0, 0.0)
    az_ref[...] = jnp.where(first, lz0, 0.0)

    def step(i, carry):
        lx, ly, lz = carry
        X = loc_ref[:, 0, :]
        Y = loc_ref[:, 1, :]
        Z = loc_ref[:, 2, :]
        dx = X - lx
        dy = Y - ly
        dz = Z - lz
        newd = (dx * dx + dy * dy) + dz * dz
        dcur = jnp.minimum(dists_ref[...], newd)
        dists_ref[...] = dcur
        maxv = jnp.max(dcur, axis=-1, keepdims=True)
        sel = jnp.min(jnp.where(dcur == maxv, colid, n), axis=-1,
                      keepdims=True)
        onehot = colid == sel
        nx = jnp.sum(jnp.where(onehot, X, 0.0), axis=-1, keepdims=True)
        ny = jnp.sum(jnp.where(onehot, Y, 0.0), axis=-1, keepdims=True)
        nz = jnp.sum(jnp.where(onehot, Z, 0.0), axis=-1, keepdims=True)
        here = colm == i
        ax_ref[...] = jnp.where(here, nx, ax_ref[...])
        ay_ref[...] = jnp.where(here, ny, ay_ref[...])
        az_ref[...] = jnp.where(here, nz, az_ref[...])
        return (nx, ny, nz)

    lax.fori_loop(1, m, step, (lx0, ly0, lz0))
    setloc_ref[:, 0, :] = ax_ref[...]
    setloc_ref[:, 1, :] = ay_ref[...]
    setloc_ref[:, 2, :] = az_ref[...]


def _fps(loc, m):
    b, d, n = loc.shape
    return pl.pallas_call(
        _fps_body,
        out_shape=jax.ShapeDtypeStruct((b, d, m), jnp.float32),
        scratch_shapes=[pltpu.VMEM((b, n), jnp.float32),
                        pltpu.VMEM((b, m), jnp.float32),
                        pltpu.VMEM((b, m), jnp.float32),
                        pltpu.VMEM((b, m), jnp.float32)],
    )(loc)


def _knn_body(qref, pref, oref, vals_ref):
    n = pref.shape[-1]
    tq = qref.shape[1]
    bi = pl.program_id(0)
    q = qref[0]
    qx = q[:, 0:1]
    qy = q[:, 1:2]
    qz = q[:, 2:3]
    px = pref[0, 0:1, :]
    py = pref[0, 1:2, :]
    pz = pref[0, 2:3, :]

    def bf(v):
        return v.astype(jnp.bfloat16).astype(jnp.float32)

    s = (bf(qx) * bf(px) + bf(qy) * bf(py)) + bf(qz) * bf(pz)
    qq = (qx * qx + qy * qy) + qz * qz
    pp = (px * px + py * py) + pz * pz
    vals_ref[...] = (qq - 2.0 * s) + pp
    colid = lax.broadcasted_iota(jnp.int32, (tq, n), 1)
    base = bi * n
    for k in range(_K):
        v = vals_ref[...]
        mv = jnp.min(v, axis=-1, keepdims=True)
        sel = jnp.min(jnp.where(v == mv, colid, n), axis=-1, keepdims=True)
        oref[0, :, k:k + 1] = sel + base
        vals_ref[...] = jnp.where(colid == sel, jnp.inf, v)


def _knn(set_loc_t, loc):
    b, m, _ = set_loc_t.shape
    n = loc.shape[-1]
    tq = min(256, m)
    return pl.pallas_call(
        _knn_body,
        grid=(b, m // tq),
        in_specs=[
            pl.BlockSpec((1, tq, 3), lambda bi, qi: (bi, qi, 0)),
            pl.BlockSpec((1, 3, n), lambda bi, qi: (bi, 0, 0)),
        ],
        out_specs=pl.BlockSpec((1, tq, _K), lambda bi, qi: (bi, qi, 0)),
        out_shape=jax.ShapeDtypeStruct((b, m, _K), jnp.int32),
        scratch_shapes=[pltpu.VMEM((tq, n), jnp.float32)],
    )(set_loc_t, loc)


def _pretab_body(ft_ref, lt_ref, w1f_ref, w1l_ref, b1_ref, o_ref):
    x = ft_ref[...]
    l = lt_ref[...]
    h = jnp.dot(x, w1f_ref[...], preferred_element_type=jnp.float32)
    h = h + l[:, 0:1] * w1l_ref[0:1, :]
    h = h + l[:, 1:2] * w1l_ref[1:2, :]
    h = h + l[:, 2:3] * w1l_ref[2:3, :]
    o_ref[...] = h + b1_ref[...]


def _pretab(feat_t, loc_t, w1f_t, w1l_t, b1):
    rows, c = feat_t.shape
    co = w1f_t.shape[1]
    tile = 1024
    return pl.pallas_call(
        _pretab_body,
        grid=(rows // tile,),
        in_specs=[
            pl.BlockSpec((tile, c), lambda i: (i, 0)),
            pl.BlockSpec((tile, 3), lambda i: (i, 0)),
            pl.BlockSpec((c, co), lambda i: (0, 0)),
            pl.BlockSpec((3, co), lambda i: (0, 0)),
            pl.BlockSpec((1, co), lambda i: (0, 0)),
        ],
        out_specs=pl.BlockSpec((tile, co), lambda i: (i, 0)),
        out_shape=jax.ShapeDtypeStruct((rows, co), jnp.float32),
    )(feat_t, loc_t, w1f_t, w1l_t, b1)


def _sc_gather(tab, idx_flat):
    rows, cf = tab.shape
    r = idx_flat.shape[0]
    nc, ns = 2, 16
    nw = nc * ns
    ch = 128
    per_w = r // nw
    n_ch = per_w // ch

    mesh = plsc.VectorSubcoreMesh(core_axis_name="c", subcore_axis_name="s",
                                  num_cores=nc, num_subcores=ns)

    @functools.partial(
        pl.kernel,
        out_type=jax.ShapeDtypeStruct((r, cf), jnp.float32),
        mesh=mesh,
        scratch_types=[
            pltpu.VMEM((ch,), jnp.int32),
            pltpu.VMEM((ch, cf), jnp.float32),
            pltpu.SemaphoreType.DMA,
        ],
    )
    def k(ftab, idx_hbm, of, idx_v, rows_v, sem1):
        wid = lax.axis_index("s") * nc + lax.axis_index("c")

        def body(j, carry):
            base = pl.multiple_of(wid * per_w + j * ch, ch)
            pltpu.sync_copy(idx_hbm.at[pl.ds(base, ch)], idx_v)
            pltpu.async_copy(ftab.at[idx_v], rows_v, sem1).wait()
            pltpu.sync_copy(rows_v, of.at[pl.ds(base, ch)])
            return carry

        lax.fori_loop(0, n_ch, body, 0)

    return k(tab, idx_flat)


def _mlp_body(gf_ref, sl_ref, w1l_ref, w2_ref, b2_ref, o_ref):
    t = sl_ref.shape[0]
    co = gf_ref.shape[-1]
    g = gf_ref[...]
    sl = sl_ref[...]
    qpl = sl[:, 0:1] * w1l_ref[0:1, :]
    qpl = qpl + sl[:, 1:2] * w1l_ref[1:2, :]
    qpl = qpl + sl[:, 2:3] * w1l_ref[2:3, :]
    qrep = jnp.reshape(jnp.broadcast_to(qpl[:, None, :], (t, _K, co)),
                       (t * _K, co))
    h = jnp.maximum(g - qrep, 0.0)
    h = jnp.dot(h, w2_ref[...], preferred_element_type=jnp.float32)
    h = jnp.maximum(h + b2_ref[...], 0.0)
    o_ref[...] = jnp.max(jnp.reshape(h, (t, _K, h.shape[-1])), axis=1)


def _mlp(gf, slq, w1l_t, w2_t, b2):
    q = slq.shape[0]
    co = gf.shape[1]
    c2 = w2_t.shape[1]
    t = min(64, q)
    return pl.pallas_call(
        _mlp_body,
        grid=(q // t,),
        in_specs=[
            pl.BlockSpec((t * _K, co), lambda i: (i, 0)),
            pl.BlockSpec((t, 3), lambda i: (i, 0)),
            pl.BlockSpec((3, co), lambda i: (0, 0)),
            pl.BlockSpec((co, c2), lambda i: (0, 0)),
            pl.BlockSpec((1, c2), lambda i: (0, 0)),
        ],
        out_specs=pl.BlockSpec((t, c2), lambda i: (i, 0)),
        out_shape=jax.ShapeDtypeStruct((q, c2), jnp.float32),
    )(gf, slq, w1l_t, w2_t, b2)


def kernel(feat, loc, W1, b1, W2, b2):
    b, c, n = feat.shape
    m = (n * 25) // 100
    c2 = W2.shape[0]

    set_loc = _fps(loc, m)
    set_loc_t = jnp.transpose(set_loc, (0, 2, 1))
    nn_idx = _knn(set_loc_t, loc)

    feat_t = jnp.transpose(feat, (0, 2, 1)).reshape(b * n, c)
    loc_t = jnp.transpose(loc, (0, 2, 1)).reshape(b * n, 3)
    w1f_t = jnp.transpose(W1[:, :c])
    w1l_t = jnp.transpose(W1[:, c:])
    tab = _pretab(feat_t, loc_t, w1f_t, w1l_t, b1[None, :])

    idx_flat = nn_idx.reshape(-1)
    gf = _sc_gather(tab, idx_flat)

    slq = set_loc_t.reshape(b * m, 3)
    w2_t = jnp.transpose(W2)
    out_flat = _mlp(gf, slq, w1l_t, w2_t, b2[None, :])
    set_feat = jnp.transpose(out_flat.reshape(b, m, c2), (0, 2, 1))
    return (set_feat, set_loc)


# speedup vs baseline: 13.5605x; 1.1177x over previous
import functools

import jax
import jax.numpy as jnp
from jax import lax
from jax.experimental import pallas as pl
from jax.experimental.pallas import tpu as pltpu
from jax.experimental.pallas import tpu_sc as plsc

_K = 32


def _pair_bcast(x, red):
    r2 = x.shape[0]
    y = red(jnp.reshape(x, (r2 // 2, 2, 1)), axis=1, keepdims=True)
    return jnp.reshape(jnp.broadcast_to(y, (r2 // 2, 2, 1)), (r2, 1))


def _fps_body(loc_ref, setloc_ref, dists_ref, ax_ref, ay_ref, az_ref):
    _, r2, nh = loc_ref.shape
    n = 2 * nh
    mh = ax_ref.shape[-1]
    half = lax.broadcasted_iota(jnp.int32, (r2, 1), 0) % 2
    colid = lax.broadcasted_iota(jnp.int32, (r2, nh), 1) + half * nh
    colm = lax.broadcasted_iota(jnp.int32, (r2, mh), 1) + half * mh
    dists_ref[...] = jnp.full((r2, nh), jnp.inf, jnp.float32)

    def first_pt(d):
        return _pair_bcast(loc_ref[d, :, 0:1] * (1 - half).astype(jnp.float32),
                           jnp.sum)

    lx0 = first_pt(0)
    ly0 = first_pt(1)
    lz0 = first_pt(2)
    first = colm == 0
    ax_ref[...] = jnp.where(first, lx0, 0.0)
    ay_ref[...] = jnp.where(first, ly0, 0.0)
    az_ref[...] = jnp.where(first, lz0, 0.0)

    def step(i, carry):
        lx, ly, lz = carry
        X = loc_ref[0]
        Y = loc_ref[1]
        Z = loc_ref[2]
        dx = X - lx
        dy = Y - ly
        dz = Z - lz
        newd = (dx * dx + dy * dy) + dz * dz
        dcur = jnp.minimum(dists_ref[...], newd)
        dists_ref[...] = dcur
        maxv = _pair_bcast(jnp.max(dcur, axis=-1, keepdims=True), jnp.max)
        sel = _pair_bcast(jnp.min(jnp.where(dcur == maxv, colid, n),
                                  axis=-1, keepdims=True), jnp.min)
        onehot = colid == sel
        nx = _pair_bcast(jnp.sum(jnp.where(onehot, X, 0.0), axis=-1,
                                 keepdims=True), jnp.sum)
        ny = _pair_bcast(jnp.sum(jnp.where(onehot, Y, 0.0), axis=-1,
                                 keepdims=True), jnp.sum)
        nz = _pair_bcast(jnp.sum(jnp.where(onehot, Z, 0.0), axis=-1,
                                 keepdims=True), jnp.sum)
        here = colm == i
        ax_ref[...] = jnp.where(here, nx, ax_ref[...])
        ay_ref[...] = jnp.where(here, ny, ay_ref[...])
        az_ref[...] = jnp.where(here, nz, az_ref[...])
        return (nx, ny, nz)

    lax.fori_loop(1, 2 * mh, step, (lx0, ly0, lz0))
    setloc_ref[0] = ax_ref[...]
    setloc_ref[1] = ay_ref[...]
    setloc_ref[2] = az_ref[...]


def _fps(loc, m):
    b, d, n = loc.shape
    locr = jnp.transpose(loc, (1, 0, 2)).reshape(d, 2 * b, n // 2)
    out = pl.pallas_call(
        _fps_body,
        out_shape=jax.ShapeDtypeStruct((d, 2 * b, m // 2), jnp.float32),
        scratch_shapes=[pltpu.VMEM((2 * b, n // 2), jnp.float32),
                        pltpu.VMEM((2 * b, m // 2), jnp.float32),
                        pltpu.VMEM((2 * b, m // 2), jnp.float32),
                        pltpu.VMEM((2 * b, m // 2), jnp.float32)],
    )(locr)
    return jnp.transpose(out.reshape(d, b, m), (1, 0, 2))


def _knn_body(qref, pref, oref, vals_ref):
    n = pref.shape[-1]
    tq = qref.shape[1]
    bi = pl.program_id(0)
    q = qref[0]
    qx = q[:, 0:1]
    qy = q[:, 1:2]
    qz = q[:, 2:3]
    px = pref[0, 0:1, :]
    py = pref[0, 1:2, :]
    pz = pref[0, 2:3, :]

    def bf(v):
        return v.astype(jnp.bfloat16).astype(jnp.float32)

    s = (bf(qx) * bf(px) + bf(qy) * bf(py)) + bf(qz) * bf(pz)
    qq = (qx * qx + qy * qy) + qz * qz
    pp = (px * px + py * py) + pz * pz
    vals_ref[...] = (qq - 2.0 * s) + pp
    colid = lax.broadcasted_iota(jnp.int32, (tq, n), 1)
    base = bi * n
    for k in range(_K):
        v = vals_ref[...]
        mv = jnp.min(v, axis=-1, keepdims=True)
        sel = jnp.min(jnp.where(v == mv, colid, n), axis=-1, keepdims=True)
        oref[0, :, k:k + 1] = sel + base
        vals_ref[...] = jnp.where(colid == sel, jnp.inf, v)


def _knn(set_loc_t, loc):
    b, m, _ = set_loc_t.shape
    n = loc.shape[-1]
    tq = min(512, m)
    return pl.pallas_call(
        _knn_body,
        grid=(b, m // tq),
        in_specs=[
            pl.BlockSpec((1, tq, 3), lambda bi, qi: (bi, qi, 0)),
            pl.BlockSpec((1, 3, n), lambda bi, qi: (bi, 0, 0)),
        ],
        out_specs=pl.BlockSpec((1, tq, _K), lambda bi, qi: (bi, qi, 0)),
        out_shape=jax.ShapeDtypeStruct((b, m, _K), jnp.int32),
        scratch_shapes=[pltpu.VMEM((tq, n), jnp.float32)],
    )(set_loc_t, loc)


def _pretab_body(ft_ref, lt_ref, w1f_ref, w1l_ref, b1_ref, o_ref):
    x = ft_ref[...]
    l = lt_ref[...]
    h = jnp.dot(x, w1f_ref[...], preferred_element_type=jnp.float32)
    h = h + l[:, 0:1] * w1l_ref[0:1, :]
    h = h + l[:, 1:2] * w1l_ref[1:2, :]
    h = h + l[:, 2:3] * w1l_ref[2:3, :]
    o_ref[...] = h + b1_ref[...]


def _pretab(feat_t, loc_t, w1f_t, w1l_t, b1):
    rows, c = feat_t.shape
    co = w1f_t.shape[1]
    tile = 1024
    return pl.pallas_call(
        _pretab_body,
        grid=(rows // tile,),
        in_specs=[
            pl.BlockSpec((tile, c), lambda i: (i, 0)),
            pl.BlockSpec((tile, 3), lambda i: (i, 0)),
            pl.BlockSpec((c, co), lambda i: (0, 0)),
            pl.BlockSpec((3, co), lambda i: (0, 0)),
            pl.BlockSpec((1, co), lambda i: (0, 0)),
        ],
        out_specs=pl.BlockSpec((tile, co), lambda i: (i, 0)),
        out_shape=jax.ShapeDtypeStruct((rows, co), jnp.float32),
    )(feat_t, loc_t, w1f_t, w1l_t, b1)


def _sc_gather(tab, idx_flat):
    rows, cf = tab.shape
    r = idx_flat.shape[0]
    nc, ns = 2, 16
    nw = nc * ns
    ch = 128
    per_w = r // nw
    n_ch = per_w // ch

    mesh = plsc.VectorSubcoreMesh(core_axis_name="c", subcore_axis_name="s",
                                  num_cores=nc, num_subcores=ns)

    @functools.partial(
        pl.kernel,
        out_type=jax.ShapeDtypeStruct((r, cf), jnp.float32),
        mesh=mesh,
        scratch_types=[
            pltpu.VMEM((ch,), jnp.int32),
            pltpu.VMEM((ch, cf), jnp.float32),
            pltpu.SemaphoreType.DMA,
        ],
    )
    def k(ftab, idx_hbm, of, idx_v, rows_v, sem1):
        wid = lax.axis_index("s") * nc + lax.axis_index("c")

        def body(j, carry):
            base = pl.multiple_of(wid * per_w + j * ch, ch)
            pltpu.sync_copy(idx_hbm.at[pl.ds(base, ch)], idx_v)
            pltpu.async_copy(ftab.at[idx_v], rows_v, sem1).wait()
            pltpu.sync_copy(rows_v, of.at[pl.ds(base, ch)])
            return carry

        lax.fori_loop(0, n_ch, body, 0)

    return k(tab, idx_flat)


def _mlp_body(gf_ref, sl_ref, w1l_ref, w2_ref, b2_ref, o_ref):
    t = sl_ref.shape[0]
    co = gf_ref.shape[-1]
    g = gf_ref[...]
    sl = sl_ref[...]
    qpl = sl[:, 0:1] * w1l_ref[0:1, :]
    qpl = qpl + sl[:, 1:2] * w1l_ref[1:2, :]
    qpl = qpl + sl[:, 2:3] * w1l_ref[2:3, :]
    qrep = jnp.reshape(jnp.broadcast_to(qpl[:, None, :], (t, _K, co)),
                       (t * _K, co))
    h = jnp.maximum(g - qrep, 0.0)
    h = jnp.dot(h, w2_ref[...], preferred_element_type=jnp.float32)
    h = jnp.maximum(h + b2_ref[...], 0.0)
    o_ref[...] = jnp.max(jnp.reshape(h, (t, _K, h.shape[-1])), axis=1)


def _mlp(gf, slq, w1l_t, w2_t, b2):
    q = slq.shape[0]
    co = gf.shape[1]
    c2 = w2_t.shape[1]
    t = min(64, q)
    return pl.pallas_call(
        _mlp_body,
        grid=(q // t,),
        in_specs=[
            pl.BlockSpec((t * _K, co), lambda i: (i, 0)),
            pl.BlockSpec((t, 3), lambda i: (i, 0)),
            pl.BlockSpec((3, co), lambda i: (0, 0)),
            pl.BlockSpec((co, c2), lambda i: (0, 0)),
            pl.BlockSpec((1, c2), lambda i: (0, 0)),
        ],
        out_specs=pl.BlockSpec((t, c2), lambda i: (i, 0)),
        out_shape=jax.ShapeDtypeStruct((q, c2), jnp.float32),
    )(gf, slq, w1l_t, w2_t, b2)


def kernel(feat, loc, W1, b1, W2, b2):
    b, c, n = feat.shape
    m = (n * 25) // 100
    c2 = W2.shape[0]

    set_loc = _fps(loc, m)
    set_loc_t = jnp.transpose(set_loc, (0, 2, 1))
    nn_idx = _knn(set_loc_t, loc)

    feat_t = jnp.transpose(feat, (0, 2, 1)).reshape(b * n, c)
    loc_t = jnp.transpose(loc, (0, 2, 1)).reshape(b * n, 3)
    w1f_t = jnp.transpose(W1[:, :c])
    w1l_t = jnp.transpose(W1[:, c:])
    tab = _pretab(feat_t, loc_t, w1f_t, w1l_t, b1[None, :])

    idx_flat = nn_idx.reshape(-1)
    gf = _sc_gather(tab, idx_flat)

    slq = set_loc_t.reshape(b * m, 3)
    w2_t = jnp.transpose(W2)
    out_flat = _mlp(gf, slq, w1l_t, w2_t, b2[None, :])
    set_feat = jnp.transpose(out_flat.reshape(b, m, c2), (0, 2, 1))
    return (set_feat, set_loc)


# speedup vs baseline: 13.8194x; 1.0191x over previous
import functools

import jax
import jax.numpy as jnp
from jax import lax
from jax.experimental import pallas as pl
from jax.experimental.pallas import tpu as pltpu
from jax.experimental.pallas import tpu_sc as plsc

_K = 32


def _pair_bcast(x, red):
    r2 = x.shape[0]
    y = red(jnp.reshape(x, (r2 // 2, 2, 1)), axis=1, keepdims=True)
    return jnp.reshape(jnp.broadcast_to(y, (r2 // 2, 2, 1)), (r2, 1))


def _fps_body(loc_ref, setloc_ref, dists_ref, ax_ref, ay_ref, az_ref):
    _, r2, nh = loc_ref.shape
    n = 2 * nh
    mh = ax_ref.shape[-1]
    half = lax.broadcasted_iota(jnp.int32, (r2, 1), 0) % 2
    colid = lax.broadcasted_iota(jnp.int32, (r2, nh), 1) + half * nh
    colm = lax.broadcasted_iota(jnp.int32, (r2, mh), 1) + half * mh
    dists_ref[...] = jnp.full((r2, nh), jnp.inf, jnp.float32)

    def first_pt(d):
        return _pair_bcast(loc_ref[d, :, 0:1] * (1 - half).astype(jnp.float32),
                           jnp.sum)

    lx0 = first_pt(0)
    ly0 = first_pt(1)
    lz0 = first_pt(2)
    first = colm == 0
    ax_ref[...] = jnp.where(first, lx0, 0.0)
    ay_ref[...] = jnp.where(first, ly0, 0.0)
    az_ref[...] = jnp.where(first, lz0, 0.0)

    def step(i, carry):
        lx, ly, lz = carry
        X = loc_ref[0]
        Y = loc_ref[1]
        Z = loc_ref[2]
        dx = X - lx
        dy = Y - ly
        dz = Z - lz
        newd = (dx * dx + dy * dy) + dz * dz
        dcur = jnp.minimum(dists_ref[...], newd)
        dists_ref[...] = dcur
        maxv = _pair_bcast(jnp.max(dcur, axis=-1, keepdims=True), jnp.max)
        sel = _pair_bcast(jnp.min(jnp.where(dcur == maxv, colid, n),
                                  axis=-1, keepdims=True), jnp.min)
        onehot = colid == sel
        sums = jnp.sum(jnp.where(onehot[None, :, :], loc_ref[...], 0.0),
                       axis=-1, keepdims=True)
        nx = _pair_bcast(sums[0], jnp.sum)
        ny = _pair_bcast(sums[1], jnp.sum)
        nz = _pair_bcast(sums[2], jnp.sum)
        here = colm == i
        ax_ref[...] = jnp.where(here, nx, ax_ref[...])
        ay_ref[...] = jnp.where(here, ny, ay_ref[...])
        az_ref[...] = jnp.where(here, nz, az_ref[...])
        return (nx, ny, nz)

    lax.fori_loop(1, 2 * mh, step, (lx0, ly0, lz0))
    setloc_ref[0] = ax_ref[...]
    setloc_ref[1] = ay_ref[...]
    setloc_ref[2] = az_ref[...]


def _fps(loc, m):
    b, d, n = loc.shape
    locr = jnp.transpose(loc, (1, 0, 2)).reshape(d, 2 * b, n // 2)
    out = pl.pallas_call(
        _fps_body,
        out_shape=jax.ShapeDtypeStruct((d, 2 * b, m // 2), jnp.float32),
        scratch_shapes=[pltpu.VMEM((2 * b, n // 2), jnp.float32),
                        pltpu.VMEM((2 * b, m // 2), jnp.float32),
                        pltpu.VMEM((2 * b, m // 2), jnp.float32),
                        pltpu.VMEM((2 * b, m // 2), jnp.float32)],
    )(locr)
    return jnp.transpose(out.reshape(d, b, m), (1, 0, 2))


def _knn_body(qref, pref, oref, vals_ref, *, row_base):
    n = pref.shape[-1]
    tq = qref.shape[1]
    q = qref[0]
    qx = q[:, 0:1]
    qy = q[:, 1:2]
    qz = q[:, 2:3]
    px = pref[0, 0:1, :]
    py = pref[0, 1:2, :]
    pz = pref[0, 2:3, :]

    def bf(v):
        return v.astype(jnp.bfloat16).astype(jnp.float32)

    s = (bf(qx) * bf(px) + bf(qy) * bf(py)) + bf(qz) * bf(pz)
    qq = (qx * qx + qy * qy) + qz * qz
    pp = (px * px + py * py) + pz * pz
    vals_ref[...] = (qq - 2.0 * s) + pp
    colid = lax.broadcasted_iota(jnp.int32, (tq, n), 1)
    for k in range(_K):
        v = vals_ref[...]
        mv = jnp.min(v, axis=-1, keepdims=True)
        sel = jnp.min(jnp.where(v == mv, colid, n), axis=-1, keepdims=True)
        oref[0, :, k:k + 1] = sel + row_base
        vals_ref[...] = jnp.where(colid == sel, jnp.inf, v)


def _knn(set_loc_t, loc, row_base):
    b, m, _ = set_loc_t.shape
    n = loc.shape[-1]
    tq = min(512, m)
    return pl.pallas_call(
        functools.partial(_knn_body, row_base=row_base),
        grid=(b, m // tq),
        in_specs=[
            pl.BlockSpec((1, tq, 3), lambda bi, qi: (bi, qi, 0)),
            pl.BlockSpec((1, 3, n), lambda bi, qi: (bi, 0, 0)),
        ],
        out_specs=pl.BlockSpec((1, tq, _K), lambda bi, qi: (bi, qi, 0)),
        out_shape=jax.ShapeDtypeStruct((b, m, _K), jnp.int32),
        scratch_shapes=[pltpu.VMEM((tq, n), jnp.float32)],
    )(set_loc_t, loc)


def _pretab_body(ft_ref, lt_ref, w1f_ref, w1l_ref, b1_ref, o_ref):
    x = ft_ref[...]
    l = lt_ref[...]
    h = jnp.dot(x, w1f_ref[...], preferred_element_type=jnp.float32)
    h = h + l[:, 0:1] * w1l_ref[0:1, :]
    h = h + l[:, 1:2] * w1l_ref[1:2, :]
    h = h + l[:, 2:3] * w1l_ref[2:3, :]
    o_ref[...] = h + b1_ref[...]


def _pretab(feat_t, loc_t, w1f_t, w1l_t, b1):
    rows, c = feat_t.shape
    co = w1f_t.shape[1]
    tile = 1024
    return pl.pallas_call(
        _pretab_body,
        grid=(rows // tile,),
        in_specs=[
            pl.BlockSpec((tile, c), lambda i: (i, 0)),
            pl.BlockSpec((tile, 3), lambda i: (i, 0)),
            pl.BlockSpec((c, co), lambda i: (0, 0)),
            pl.BlockSpec((3, co), lambda i: (0, 0)),
            pl.BlockSpec((1, co), lambda i: (0, 0)),
        ],
        out_specs=pl.BlockSpec((tile, co), lambda i: (i, 0)),
        out_shape=jax.ShapeDtypeStruct((rows, co), jnp.float32),
    )(feat_t, loc_t, w1f_t, w1l_t, b1)


def _sc_gather(tab, idx_flat):
    rows, cf = tab.shape
    r = idx_flat.shape[0]
    nc, ns = 2, 16
    nw = nc * ns
    ch = 128
    per_w = r // nw
    n_ch = per_w // ch

    mesh = plsc.VectorSubcoreMesh(core_axis_name="c", subcore_axis_name="s",
                                  num_cores=nc, num_subcores=ns)

    @functools.partial(
        pl.kernel,
        out_type=jax.ShapeDtypeStruct((r, cf), jnp.float32),
        mesh=mesh,
        scratch_types=[
            pltpu.VMEM((ch,), jnp.int32),
            pltpu.VMEM((ch, cf), jnp.float32),
            pltpu.SemaphoreType.DMA,
        ],
    )
    def k(ftab, idx_hbm, of, idx_v, rows_v, sem1):
        wid = lax.axis_index("s") * nc + lax.axis_index("c")

        def body(j, carry):
            base = pl.multiple_of(wid * per_w + j * ch, ch)
            pltpu.sync_copy(idx_hbm.at[pl.ds(base, ch)], idx_v)
            pltpu.async_copy(ftab.at[idx_v], rows_v, sem1).wait()
            pltpu.sync_copy(rows_v, of.at[pl.ds(base, ch)])
            return carry

        lax.fori_loop(0, n_ch, body, 0)

    return k(tab, idx_flat)


def _mlp_body(gf_ref, sl_ref, w1l_ref, w2_ref, b2_ref, o_ref):
    t = sl_ref.shape[0]
    co = gf_ref.shape[-1]
    g = gf_ref[...]
    sl = sl_ref[...]
    qpl = sl[:, 0:1] * w1l_ref[0:1, :]
    qpl = qpl + sl[:, 1:2] * w1l_ref[1:2, :]
    qpl = qpl + sl[:, 2:3] * w1l_ref[2:3, :]
    qrep = jnp.reshape(jnp.broadcast_to(qpl[:, None, :], (t, _K, co)),
                       (t * _K, co))
    h = jnp.maximum(g - qrep, 0.0)
    h = jnp.dot(h, w2_ref[...], preferred_element_type=jnp.float32)
    h = jnp.maximum(h + b2_ref[...], 0.0)
    o_ref[...] = jnp.max(jnp.reshape(h, (t, _K, h.shape[-1])), axis=1)


def _mlp(gf, slq, w1l_t, w2_t, b2):
    q = slq.shape[0]
    co = gf.shape[1]
    c2 = w2_t.shape[1]
    t = min(64, q)
    return pl.pallas_call(
        _mlp_body,
        grid=(q // t,),
        in_specs=[
            pl.BlockSpec((t * _K, co), lambda i: (i, 0)),
            pl.BlockSpec((t, 3), lambda i: (i, 0)),
            pl.BlockSpec((3, co), lambda i: (0, 0)),
            pl.BlockSpec((co, c2), lambda i: (0, 0)),
            pl.BlockSpec((1, c2), lambda i: (0, 0)),
        ],
        out_specs=pl.BlockSpec((t, c2), lambda i: (i, 0)),
        out_shape=jax.ShapeDtypeStruct((q, c2), jnp.float32),
    )(gf, slq, w1l_t, w2_t, b2)


def kernel(feat, loc, W1, b1, W2, b2):
    b, c, n = feat.shape
    m = (n * 25) // 100
    c2 = W2.shape[0]

    set_loc = _fps(loc, m)
    set_loc_t = jnp.transpose(set_loc, (0, 2, 1))

    feat_t = jnp.transpose(feat, (0, 2, 1)).reshape(b * n, c)
    loc_t = jnp.transpose(loc, (0, 2, 1)).reshape(b * n, 3)
    w1f_t = jnp.transpose(W1[:, :c])
    w1l_t = jnp.transpose(W1[:, c:])
    tab = _pretab(feat_t, loc_t, w1f_t, w1l_t, b1[None, :])
    w2_t = jnp.transpose(W2)
    b2r = b2[None, :]

    outs = []
    for bi in range(b):
        nn_bi = _knn(set_loc_t[bi:bi + 1], loc[bi:bi + 1], bi * n)
        gf = _sc_gather(tab, nn_bi.reshape(-1))
        slq = set_loc_t[bi]
        outs.append(_mlp(gf, slq, w1l_t, w2_t, b2r))
    set_feat = jnp.transpose(jnp.stack(outs, axis=0), (0, 2, 1))
    return (set_feat, set_loc)
